# Initial kernel scaffold; baseline (speedup 1.0000x reference)
import jax
import jax.numpy as jnp
from jax.experimental import pallas as pl


def kernel(h, edge_index):
    raise NotImplementedError("write your pallas kernel here")


# speedup vs baseline: 211.8988x; 211.8988x over previous
import functools

import jax
import jax.numpy as jnp
from jax import lax
from jax.experimental import pallas as pl
from jax.experimental.pallas import tpu as pltpu
from jax.experimental.pallas import tpu_sc as plsc

NC = 2
NS = 16
NW = NC * NS
CH = 7168

_mesh = plsc.VectorSubcoreMesh(core_axis_name="c", subcore_axis_name="s")


def _sc_degrees(n_pad, e_pad):
  nch = e_pad // (NW * CH)
  ew = e_pad // NW
  sl_sz = n_pad // NS

  @functools.partial(
      pl.kernel,
      mesh=_mesh,
      out_type=[
          jax.ShapeDtypeStruct((NC, n_pad), jnp.float32),
          jax.ShapeDtypeStruct((NC, n_pad), jnp.float32),
      ],
      scratch_types=[
          pltpu.VMEM((CH,), jnp.int32),
          pltpu.VMEM((CH,), jnp.int32),
          pltpu.VMEM((CH,), jnp.float32),
          pltpu.VMEM_SHARED((n_pad,), jnp.float32),
          pltpu.VMEM_SHARED((n_pad,), jnp.float32),
      ],
  )
  def deg_kernel(src_hbm, dst_hbm, zeros_hbm, ones_hbm,
                 dego_hbm, degi_hbm,
                 src_buf, dst_buf, ones_buf, dego_sh, degi_sh):
    c = lax.axis_index("c")
    s = lax.axis_index("s")
    sl = pl.ds(s * sl_sz, sl_sz)
    pltpu.sync_copy(zeros_hbm.at[sl], dego_sh.at[sl])
    pltpu.sync_copy(zeros_hbm.at[sl], degi_sh.at[sl])
    pltpu.sync_copy(ones_hbm, ones_buf)
    plsc.subcore_barrier()

    base = (c * NS + s) * ew

    def body(i, _):
      off = base + i * CH
      pltpu.sync_copy(src_hbm.at[pl.ds(off, CH)], src_buf)
      pltpu.sync_copy(dst_hbm.at[pl.ds(off, CH)], dst_buf)
      pltpu.sync_copy(ones_buf, dego_sh.at[src_buf], add=True)
      pltpu.sync_copy(ones_buf, degi_sh.at[dst_buf], add=True)
      return 0

    lax.fori_loop(0, nch, body, 0)
    plsc.subcore_barrier()
    pltpu.sync_copy(dego_sh.at[sl], dego_hbm.at[c].at[sl])
    pltpu.sync_copy(degi_sh.at[sl], degi_hbm.at[c].at[sl])

  return deg_kernel


def _sc_layer(n_pad, e_pad):
  nch = e_pad // (NW * CH)
  ew = e_pad // NW
  sl_sz = n_pad // NS

  @functools.partial(
      pl.kernel,
      mesh=_mesh,
      out_type=jax.ShapeDtypeStruct((NC, n_pad), jnp.float32),
      scratch_types=[
          pltpu.VMEM((CH,), jnp.int32),
          pltpu.VMEM((CH,), jnp.int32),
          pltpu.VMEM((CH,), jnp.float32),
          pltpu.VMEM_SHARED((n_pad,), jnp.float32),
          pltpu.VMEM_SHARED((n_pad,), jnp.float32),
      ],
  )
  def layer_kernel(u_hbm, src_hbm, dst_hbm, zeros_hbm,
                   acc_hbm,
                   src_buf, dst_buf, val_buf, u_sh, acc_sh):
    c = lax.axis_index("c")
    s = lax.axis_index("s")
    sl = pl.ds(s * sl_sz, sl_sz)
    pltpu.sync_copy(u_hbm.at[sl], u_sh.at[sl])
    pltpu.sync_copy(zeros_hbm.at[sl], acc_sh.at[sl])
    plsc.subcore_barrier()

    base = (c * NS + s) * ew

    def body(i, _):
      off = base + i * CH
      pltpu.sync_copy(src_hbm.at[pl.ds(off, CH)], src_buf)
      pltpu.sync_copy(dst_hbm.at[pl.ds(off, CH)], dst_buf)
      pltpu.sync_copy(u_sh.at[src_buf], val_buf)
      pltpu.sync_copy(val_buf, acc_sh.at[dst_buf], add=True)
      return 0

    lax.fori_loop(0, nch, body, 0)
    plsc.subcore_barrier()
    pltpu.sync_copy(acc_sh.at[sl], acc_hbm.at[c].at[sl])

  return layer_kernel


def _tc_norms(r):
  def body(dego_ref, degi_ref, h_ref, ns_ref, nd_ref, u0_ref):
    od = dego_ref[0] + dego_ref[1]
    idg = degi_ref[0] + degi_ref[1]
    ns = lax.rsqrt(jnp.maximum(od, 1.0))
    nd = lax.rsqrt(jnp.maximum(idg, 1.0))
    ns_ref[...] = ns
    nd_ref[...] = nd
    u0_ref[...] = h_ref[...] * ns

  return pl.pallas_call(
      body,
      out_shape=[
          jax.ShapeDtypeStruct((r, 128), jnp.float32),
          jax.ShapeDtypeStruct((r, 128), jnp.float32),
          jax.ShapeDtypeStruct((r, 128), jnp.float32),
      ],
  )


def _tc_merge(r):
  def body(acc_ref, ns_ref, nd_ref, u_ref, c5_ref):
    v = (acc_ref[0] + acc_ref[1]) * nd_ref[...]
    u_ref[...] = v * ns_ref[...]
    c5_ref[...] = jnp.sum(v * v).reshape(1, 1)

  return pl.pallas_call(
      body,
      out_shape=[
          jax.ShapeDtypeStruct((r, 128), jnp.float32),
          jax.ShapeDtypeStruct((1, 1), jnp.float32),
      ],
  )


def kernel(h, edge_index):
  n = h.shape[0]
  e = edge_index.shape[1]
  n_pad = ((n + 2047) // 2048) * 2048
  r = n_pad // 128
  e_pad = ((e + NW * CH - 1) // (NW * CH)) * (NW * CH)

  src = edge_index[0].astype(jnp.int32)
  dst = edge_index[1].astype(jnp.int32)
  if e_pad != e:
    pad = jnp.full((e_pad - e,), n_pad - 1, dtype=jnp.int32)
    src = jnp.concatenate([src, pad])
    dst = jnp.concatenate([dst, pad])
  hv = jnp.pad(h[:, 0], (0, n_pad - n))
  zeros = jnp.zeros((n_pad,), jnp.float32)
  ones = jnp.ones((CH,), jnp.float32)

  dego, degi = _sc_degrees(n_pad, e_pad)(src, dst, zeros, ones)
  ns, nd, u = _tc_norms(r)(
      dego.reshape(NC, r, 128), degi.reshape(NC, r, 128), hv.reshape(r, 128)
  )

  layer = _sc_layer(n_pad, e_pad)
  merge = _tc_merge(r)
  c5s = []
  for _ in range(3):
    accp = layer(u.reshape(-1), src, dst, zeros)
    u, c5 = merge(accp.reshape(NC, r, 128), ns, nd)
    c5s.append(c5[0, 0])
  return jnp.stack(c5s)
